# baseline (speedup 1.0000x reference)
import functools

import jax
import jax.numpy as jnp
from jax import lax
from jax.experimental import pallas as pl
from jax.experimental.pallas import tpu as pltpu
from jax.experimental.pallas import tpu_sc as plsc

N_NODES = 10000
N_EDGES = 320000
D = 128
DH = D // 2

NC = 2
NS = 16
NW = NC * NS

CHUNK = 64
NBUF = 5
E_PAD = 327680
E_PER_TILE = E_PAD // NS
CHUNKS_PER_TILE = E_PER_TILE // CHUNK
IDX_PARTS = 8
IDX_PART = CHUNKS_PER_TILE // IDX_PARTS
N_PAD = 10240
ROWS_PER_TILE = N_PAD // NS

_mesh = plsc.VectorSubcoreMesh(
    core_axis_name="c", subcore_axis_name="s", num_cores=NC, num_subcores=NS
)


@functools.partial(
    pl.kernel,
    out_type=jax.ShapeDtypeStruct((NC, N_PAD, DH), jnp.float32),
    mesh=_mesh,
    scratch_types=[
        pltpu.VMEM_SHARED((N_PAD, DH), jnp.float32),
        pltpu.VMEM_SHARED((N_PAD, DH), jnp.float32),
        pltpu.VMEM((IDX_PART, CHUNK), jnp.int32),
        pltpu.VMEM((IDX_PART, CHUNK), jnp.int32),
        pltpu.VMEM((NBUF, CHUNK, DH), jnp.float32),
        [pltpu.SemaphoreType.DMA] * NBUF,
        [pltpu.SemaphoreType.DMA] * NBUF,
    ],
    compiler_params=pltpu.CompilerParams(use_tc_tiling_on_sc=False),
)
def _sc_aggregate(x_hbm, src_hbm, dst_hbm, out_hbm, acc, x_sp, src_v, dst_v,
                  rows_v, gsems, ssems):
    cid = lax.axis_index("c")
    sid = lax.axis_index("s")

    def _zero_row(r, _):
        for j in range(DH // 16):
            rows_v[0, r, pl.ds(j * 16, 16)] = jnp.zeros((16,), jnp.float32)
        return 0

    with jax.named_scope("p_zerobuf"):
        lax.fori_loop(0, CHUNK, _zero_row, 0)

    base_rows = sid * ROWS_PER_TILE
    col0 = cid * DH
    with jax.named_scope("p_stage_x"):
        pltpu.sync_copy(
            x_hbm.at[pl.ds(base_rows, 400), pl.ds(col0, DH)],
            x_sp.at[pl.ds(base_rows, 400)],
        )

    @pl.when(sid < NS - 1)
    def _():
        pltpu.sync_copy(
            x_hbm.at[pl.ds(base_rows + 400, ROWS_PER_TILE - 400), pl.ds(col0, DH)],
            x_sp.at[pl.ds(base_rows + 400, ROWS_PER_TILE - 400)],
        )

    @pl.when(sid == NS - 1)
    def _():
        for r0 in range(N_NODES, N_PAD, CHUNK):
            n = min(CHUNK, N_PAD - r0)
            pltpu.sync_copy(rows_v.at[0, pl.ds(0, n)], x_sp.at[pl.ds(r0, n)])

    for r0 in range(0, ROWS_PER_TILE, CHUNK):
        pltpu.sync_copy(
            rows_v.at[0, pl.ds(0, CHUNK)], acc.at[pl.ds(base_rows + r0, CHUNK)]
        )

    with jax.named_scope("p_barrier1"):
        plsc.subcore_barrier()

    base_chunk = sid * CHUNKS_PER_TILE
    for h in range(IDX_PARTS):
        hbase = base_chunk + h * IDX_PART
        pltpu.sync_copy(src_hbm.at[pl.ds(hbase, IDX_PART)], src_v)
        pltpu.sync_copy(dst_hbm.at[pl.ds(hbase, IDX_PART)], dst_v)

        for b in range(NBUF - 1):
            pltpu.async_copy(x_sp.at[src_v.at[b]], rows_v.at[b], gsems[b])

        def _edge_group(g, _):
            c0 = g * NBUF
            for b in range(NBUF):
                c = c0 + b
                bp = (b + NBUF - 1) % NBUF
                pltpu.make_async_copy(
                    x_sp.at[src_v.at[c]], rows_v.at[b], gsems[b]
                ).wait()

                @pl.when(c > 0)
                def _():
                    pltpu.make_async_copy(
                        rows_v.at[bp], acc.at[dst_v.at[c - 1]], ssems[bp]
                    ).wait()

                pltpu.async_copy(rows_v.at[b], acc.at[dst_v.at[c]], ssems[b], add=True)
                nxt = c + NBUF - 1

                @pl.when(nxt < IDX_PART)
                def _():
                    pltpu.async_copy(x_sp.at[src_v.at[nxt]], rows_v.at[bp], gsems[bp])

            return 0

        with jax.named_scope("p_edge_loop"):
            lax.fori_loop(0, IDX_PART // NBUF, _edge_group, 0)

        pltpu.make_async_copy(
            rows_v.at[(IDX_PART - 1) % NBUF],
            acc.at[dst_v.at[IDX_PART - 1]],
            ssems[(IDX_PART - 1) % NBUF],
        ).wait()

    with jax.named_scope("p_barrier2"):
        plsc.subcore_barrier()

    with jax.named_scope("p_writeout"):
        pltpu.sync_copy(
            acc.at[pl.ds(base_rows, ROWS_PER_TILE)],
            out_hbm.at[cid, pl.ds(base_rows, ROWS_PER_TILE)],
        )


_BLK = 1000


def _combine_matmul_body(p_ref, w_ref, b_ref, o_ref):
    p = jnp.concatenate([p_ref[0], p_ref[1]], axis=-1)
    o_ref[...] = (
        jnp.dot(p, w_ref[...], preferred_element_type=jnp.float32) + b_ref[...]
    )


def _combine_matmul(partials, W, b2d):
    return pl.pallas_call(
        _combine_matmul_body,
        grid=(N_NODES // _BLK,),
        in_specs=[
            pl.BlockSpec((NC, _BLK, DH), lambda i: (0, i, 0)),
            pl.BlockSpec((D, D), lambda i: (0, 0)),
            pl.BlockSpec((1, D), lambda i: (0, 0)),
        ],
        out_specs=pl.BlockSpec((_BLK, D), lambda i: (i, 0)),
        out_shape=jax.ShapeDtypeStruct((N_NODES, D), jnp.float32),
    )(partials, W, b2d)


def kernel(x, edge_index, W, b):
    src = edge_index[0].astype(jnp.int32)
    dst = edge_index[1].astype(jnp.int32)

    pad = E_PAD - N_EDGES
    pad_idx = jnp.full((pad,), N_NODES, dtype=jnp.int32)
    src2d = jnp.concatenate([src, pad_idx]).reshape(E_PAD // CHUNK, CHUNK)
    dst2d = jnp.concatenate([dst, pad_idx]).reshape(E_PAD // CHUNK, CHUNK)

    partials = _sc_aggregate(x, src2d, dst2d)
    out = _combine_matmul(partials, W, b.reshape(1, D))
    return (out,)


# speedup vs baseline: 1.0916x; 1.0916x over previous
import functools

import jax
import jax.numpy as jnp
from jax import lax
from jax.experimental import pallas as pl
from jax.experimental.pallas import tpu as pltpu
from jax.experimental.pallas import tpu_sc as plsc

N_NODES = 10000
N_EDGES = 320000
D = 128
DH = D // 2

NC = 2
NS = 16
NW = NC * NS

CHUNK = 64
NBUF = 4
E_PAD = 327680
E_PER_TILE = E_PAD // NS
CHUNKS_PER_TILE = E_PER_TILE // CHUNK
IDX_PARTS = 8
IDX_PART = CHUNKS_PER_TILE // IDX_PARTS
N_PAD = 10240
ROWS_PER_TILE = N_PAD // NS

_mesh = plsc.VectorSubcoreMesh(
    core_axis_name="c", subcore_axis_name="s", num_cores=NC, num_subcores=NS
)


@functools.partial(
    pl.kernel,
    out_type=jax.ShapeDtypeStruct((NC, N_PAD, DH), jnp.float32),
    mesh=_mesh,
    scratch_types=[
        pltpu.VMEM_SHARED((N_PAD, DH), jnp.float32),
        pltpu.VMEM_SHARED((N_PAD, DH), jnp.float32),
        pltpu.VMEM((IDX_PART, CHUNK), jnp.int32),
        pltpu.VMEM((IDX_PART, CHUNK), jnp.int32),
        pltpu.VMEM((NBUF, CHUNK, DH), jnp.float32),
        [pltpu.SemaphoreType.DMA] * NBUF,
        [pltpu.SemaphoreType.DMA] * NBUF,
    ],
    compiler_params=pltpu.CompilerParams(use_tc_tiling_on_sc=False),
)
def _sc_aggregate(x_hbm, src_hbm, dst_hbm, out_hbm, acc, x_sp, src_v, dst_v,
                  rows_v, gsems, ssems):
    cid = lax.axis_index("c")
    sid = lax.axis_index("s")

    def _zero_row(r, _):
        for j in range(DH // 16):
            rows_v[0, r, pl.ds(j * 16, 16)] = jnp.zeros((16,), jnp.float32)
        return 0

    with jax.named_scope("p_zerobuf"):
        lax.fori_loop(0, CHUNK, _zero_row, 0)

    base_rows = sid * ROWS_PER_TILE
    col0 = cid * DH
    with jax.named_scope("p_stage_x"):
        pltpu.sync_copy(
            x_hbm.at[pl.ds(base_rows, 400), pl.ds(col0, DH)],
            x_sp.at[pl.ds(base_rows, 400)],
        )

    @pl.when(sid < NS - 1)
    def _():
        pltpu.sync_copy(
            x_hbm.at[pl.ds(base_rows + 400, ROWS_PER_TILE - 400), pl.ds(col0, DH)],
            x_sp.at[pl.ds(base_rows + 400, ROWS_PER_TILE - 400)],
        )

    @pl.when(sid == NS - 1)
    def _():
        for r0 in range(N_NODES, N_PAD, CHUNK):
            n = min(CHUNK, N_PAD - r0)
            pltpu.sync_copy(rows_v.at[0, pl.ds(0, n)], x_sp.at[pl.ds(r0, n)])

    for r0 in range(0, ROWS_PER_TILE, CHUNK):
        pltpu.sync_copy(
            rows_v.at[0, pl.ds(0, CHUNK)], acc.at[pl.ds(base_rows + r0, CHUNK)]
        )

    with jax.named_scope("p_barrier1"):
        plsc.subcore_barrier()

    base_chunk = sid * CHUNKS_PER_TILE
    for h in range(IDX_PARTS):
        hbase = base_chunk + h * IDX_PART
        pltpu.sync_copy(src_hbm.at[pl.ds(hbase, IDX_PART)], src_v)
        pltpu.sync_copy(dst_hbm.at[pl.ds(hbase, IDX_PART)], dst_v)

        for b in range(NBUF - 2):
            pltpu.async_copy(x_sp.at[src_v.at[b]], rows_v.at[b], gsems[b])

        def _edge_group(g, _):
            c0 = g * NBUF
            for b in range(NBUF):
                c = c0 + b
                bp = (b + NBUF - 2) % NBUF
                pltpu.make_async_copy(
                    x_sp.at[src_v.at[c]], rows_v.at[b], gsems[b]
                ).wait()

                @pl.when(c > 1)
                def _():
                    pltpu.make_async_copy(
                        rows_v.at[bp], acc.at[dst_v.at[c - 2]], ssems[bp]
                    ).wait()

                pltpu.async_copy(rows_v.at[b], acc.at[dst_v.at[c]], ssems[b], add=True)
                nxt = c + NBUF - 2

                @pl.when(nxt < IDX_PART)
                def _():
                    pltpu.async_copy(x_sp.at[src_v.at[nxt]], rows_v.at[bp], gsems[bp])

            return 0

        with jax.named_scope("p_edge_loop"):
            lax.fori_loop(0, IDX_PART // NBUF, _edge_group, 0)

        for cl in (IDX_PART - 2, IDX_PART - 1):
            pltpu.make_async_copy(
                rows_v.at[cl % NBUF],
                acc.at[dst_v.at[cl]],
                ssems[cl % NBUF],
            ).wait()

    with jax.named_scope("p_barrier2"):
        plsc.subcore_barrier()

    with jax.named_scope("p_writeout"):
        pltpu.sync_copy(
            acc.at[pl.ds(base_rows, ROWS_PER_TILE)],
            out_hbm.at[cid, pl.ds(base_rows, ROWS_PER_TILE)],
        )


_BLK = 1000


def _combine_matmul_body(p_ref, w_ref, b_ref, o_ref):
    p = jnp.concatenate([p_ref[0], p_ref[1]], axis=-1)
    o_ref[...] = (
        jnp.dot(p, w_ref[...], preferred_element_type=jnp.float32) + b_ref[...]
    )


def _combine_matmul(partials, W, b2d):
    return pl.pallas_call(
        _combine_matmul_body,
        grid=(N_NODES // _BLK,),
        in_specs=[
            pl.BlockSpec((NC, _BLK, DH), lambda i: (0, i, 0)),
            pl.BlockSpec((D, D), lambda i: (0, 0)),
            pl.BlockSpec((1, D), lambda i: (0, 0)),
        ],
        out_specs=pl.BlockSpec((_BLK, D), lambda i: (i, 0)),
        out_shape=jax.ShapeDtypeStruct((N_NODES, D), jnp.float32),
    )(partials, W, b2d)


def kernel(x, edge_index, W, b):
    src = edge_index[0].astype(jnp.int32)
    dst = edge_index[1].astype(jnp.int32)

    pad = E_PAD - N_EDGES
    pad_idx = jnp.full((pad,), N_NODES, dtype=jnp.int32)
    src2d = jnp.concatenate([src, pad_idx]).reshape(E_PAD // CHUNK, CHUNK)
    dst2d = jnp.concatenate([dst, pad_idx]).reshape(E_PAD // CHUNK, CHUNK)

    partials = _sc_aggregate(x, src2d, dst2d)
    out = _combine_matmul(partials, W, b.reshape(1, D))
    return (out,)
